# baseline (speedup 1.0000x reference)
import functools

import jax
import jax.numpy as jnp
from jax import lax
from jax.experimental import pallas as pl
from jax.experimental.pallas import tpu as pltpu
from jax.experimental.pallas import tpu_sc as plsc

DIM = 2048
N_CHANNELS = 32
TABLE = 2 * DIM - 1
NSHIFT = 128
TW = TABLE - 1
SW = 3968
N_Q = 8
N_K = 8
N_V = 16
CHUNK = 32
NCHUNKS = NSHIFT // CHUNK
NBLOCKS = DIM // NSHIFT


PVAR = 9
SLABW = TABLE - 1

N_R8 = NSHIFT // 8
N_T = SW // 128


@functools.partial(
    pl.kernel,
    out_type=jax.ShapeDtypeStruct((N_CHANNELS, N_R8, N_T, 8, 128), jnp.float32),
    mesh=plsc.VectorSubcoreMesh(core_axis_name="c", subcore_axis_name="s"),
    compiler_params=pltpu.CompilerParams(use_tc_tiling_on_sc=False),
    scratch_types=[
        pltpu.VMEM((PVAR, SLABW), jnp.float32),
        pltpu.SemaphoreType.DMA,
    ],
)
def _sc_table(slab_hbm, t5_hbm, slabv, sem):
    wid = lax.axis_index("s") * 2 + lax.axis_index("c")
    pltpu.sync_copy(slab_hbm.at[wid], slabv)
    window = 8

    def body(i, carry):
        r8 = i // N_T
        t = i - r8 * N_T
        off = pl.multiple_of(120 - 8 * r8 + 128 * t, 8)
        pltpu.make_async_copy(
            slabv.at[pl.ds(1, 8), pl.ds(off, 128)],
            t5_hbm.at[wid, r8, t],
            sem,
        ).start()

        @pl.when(i >= window)
        def _():
            pltpu.make_async_copy(
                slabv.at[pl.ds(1, 8), pl.ds(0, 128)],
                t5_hbm.at[wid, 0, 0],
                sem,
            ).wait()

        return carry

    lax.fori_loop(0, N_R8 * N_T, body, 0)

    def drain(i, carry):
        pltpu.make_async_copy(
            slabv.at[pl.ds(1, 8), pl.ds(0, 128)],
            t5_hbm.at[wid, 0, 0],
            sem,
        ).wait()
        return carry

    lax.fori_loop(0, window, drain, 0)


def _emit_rows(dst_hbm, c_local, table_hbm, c_global, chunk, sem):
    for q in range(NCHUNKS):
        pltpu.sync_copy(
            table_hbm.at[c_global, pl.ds(q * CHUNK, CHUNK), pl.ds(0, SW)],
            chunk,
        )
        for m in range(NBLOCKS):
            a = (NBLOCKS - 1 - m) * NSHIFT
            pltpu.make_async_copy(
                chunk.at[:, pl.ds(a, DIM)],
                dst_hbm.at[c_local, pl.ds(m * NSHIFT + q * CHUNK, CHUNK)],
                sem,
            ).start()
        for m in range(NBLOCKS):
            pltpu.make_async_copy(
                chunk.at[:, pl.ds(0, DIM)],
                dst_hbm.at[c_local, pl.ds(0, CHUNK)],
                sem,
            ).wait()


@functools.partial(
    pl.kernel,
    out_type=(
        jax.ShapeDtypeStruct((N_Q, DIM, DIM), jnp.float32),
        jax.ShapeDtypeStruct((N_K, DIM, DIM), jnp.float32),
        jax.ShapeDtypeStruct((N_V, DIM, DIM), jnp.float32),
    ),
    mesh=plsc.VectorSubcoreMesh(core_axis_name="c", subcore_axis_name="s"),
    scratch_types=[
        pltpu.VMEM((CHUNK, SW), jnp.float32),
        pltpu.SemaphoreType.DMA,
    ],
)
def _sc_expand(table_hbm, q_hbm, k_hbm, v_hbm, chunk, sem):
    wid = lax.axis_index("s") * 2 + lax.axis_index("c")

    @pl.when(wid < N_Q)
    def _():
        _emit_rows(q_hbm, wid, table_hbm, wid, chunk, sem)

    @pl.when((wid >= N_Q) & (wid < N_Q + N_K))
    def _():
        _emit_rows(k_hbm, wid - N_Q, table_hbm, wid, chunk, sem)

    @pl.when(wid >= N_Q + N_K)
    def _():
        _emit_rows(v_hbm, wid - (N_Q + N_K), table_hbm, wid, chunk, sem)


def kernel(relative, flatten_index):
    del flatten_index
    rev = relative[:, ::-1]
    sflat = jnp.broadcast_to(rev[:, None, :], (N_CHANNELS, PVAR, TABLE))
    sflat = sflat.reshape(N_CHANNELS, PVAR * TABLE)
    slab = sflat[:, 8 : 8 + PVAR * SLABW].reshape(N_CHANNELS, PVAR, SLABW)
    t5 = _sc_table(slab)
    table = t5.transpose(0, 1, 3, 2, 4).reshape(N_CHANNELS, NSHIFT, SW)
    return _sc_expand(table)


# speedup vs baseline: 1.0181x; 1.0181x over previous
import functools

import jax
import jax.numpy as jnp
from jax import lax
from jax.experimental import pallas as pl
from jax.experimental.pallas import tpu as pltpu
from jax.experimental.pallas import tpu_sc as plsc

DIM = 2048
N_CHANNELS = 32
TABLE = 2 * DIM - 1
NSHIFT = 128
TW = TABLE - 1
SW = 3968
N_Q = 8
N_K = 8
N_V = 16
CHUNK = 32
NCHUNKS = NSHIFT // CHUNK
NBLOCKS = DIM // NSHIFT


PVAR = 9
SLABW = TABLE + 17

N_R8 = NSHIFT // 8
N_T = SW // 128


@functools.partial(
    pl.kernel,
    out_type=jax.ShapeDtypeStruct((N_CHANNELS, N_R8, N_T, 8, 128), jnp.float32),
    mesh=plsc.VectorSubcoreMesh(core_axis_name="c", subcore_axis_name="s"),
    compiler_params=pltpu.CompilerParams(use_tc_tiling_on_sc=False),
    scratch_types=[
        pltpu.VMEM((PVAR, SLABW), jnp.float32),
        pltpu.SemaphoreType.DMA,
    ],
)
def _sc_table(slab_hbm, t5_hbm, slabv, sem):
    wid = lax.axis_index("s") * 2 + lax.axis_index("c")
    pltpu.sync_copy(slab_hbm.at[wid], slabv)
    window = 8

    def body(i, carry):
        r8 = i // N_T
        t = i - r8 * N_T
        off = pl.multiple_of(128 - 8 * r8 + 128 * t, 8)
        pltpu.make_async_copy(
            slabv.at[pl.ds(1, 8), pl.ds(off, 128)],
            t5_hbm.at[wid, r8, t],
            sem,
        ).start()

        @pl.when(i >= window)
        def _():
            pltpu.make_async_copy(
                slabv.at[pl.ds(1, 8), pl.ds(0, 128)],
                t5_hbm.at[wid, 0, 0],
                sem,
            ).wait()

        return carry

    lax.fori_loop(0, N_R8 * N_T, body, 0)

    def drain(i, carry):
        pltpu.make_async_copy(
            slabv.at[pl.ds(1, 8), pl.ds(0, 128)],
            t5_hbm.at[wid, 0, 0],
            sem,
        ).wait()
        return carry

    lax.fori_loop(0, window, drain, 0)


def _emit_rows(dst_hbm, c_local, table_hbm, c_global, chunk, sem):
    for q in range(NCHUNKS):
        pltpu.sync_copy(
            table_hbm.at[c_global, pl.ds(q * CHUNK, CHUNK), pl.ds(0, SW)],
            chunk,
        )
        for m in range(NBLOCKS):
            a = (NBLOCKS - 1 - m) * NSHIFT
            pltpu.make_async_copy(
                chunk.at[:, pl.ds(a, DIM)],
                dst_hbm.at[c_local, pl.ds(m * NSHIFT + q * CHUNK, CHUNK)],
                sem,
            ).start()
        for m in range(NBLOCKS):
            pltpu.make_async_copy(
                chunk.at[:, pl.ds(0, DIM)],
                dst_hbm.at[c_local, pl.ds(0, CHUNK)],
                sem,
            ).wait()


@functools.partial(
    pl.kernel,
    out_type=(
        jax.ShapeDtypeStruct((N_Q, DIM, DIM), jnp.float32),
        jax.ShapeDtypeStruct((N_K, DIM, DIM), jnp.float32),
        jax.ShapeDtypeStruct((N_V, DIM, DIM), jnp.float32),
    ),
    mesh=plsc.VectorSubcoreMesh(core_axis_name="c", subcore_axis_name="s"),
    scratch_types=[
        pltpu.VMEM((CHUNK, SW), jnp.float32),
        pltpu.SemaphoreType.DMA,
    ],
)
def _sc_expand(table_hbm, q_hbm, k_hbm, v_hbm, chunk, sem):
    wid = lax.axis_index("s") * 2 + lax.axis_index("c")

    @pl.when(wid < N_Q)
    def _():
        _emit_rows(q_hbm, wid, table_hbm, wid, chunk, sem)

    @pl.when((wid >= N_Q) & (wid < N_Q + N_K))
    def _():
        _emit_rows(k_hbm, wid - N_Q, table_hbm, wid, chunk, sem)

    @pl.when(wid >= N_Q + N_K)
    def _():
        _emit_rows(v_hbm, wid - (N_Q + N_K), table_hbm, wid, chunk, sem)


def kernel(relative, flatten_index):
    del flatten_index
    rev = relative[:, ::-1]
    slab = jnp.stack(
        [jnp.pad(rev, ((0, 0), (p, 17 - p))) for p in range(PVAR)], axis=1
    )
    t5 = _sc_table(slab)
    table = t5.transpose(0, 1, 3, 2, 4).reshape(N_CHANNELS, NSHIFT, SW)
    return _sc_expand(table)
